# Initial kernel scaffold; baseline (speedup 1.0000x reference)
import jax
import jax.numpy as jnp
from jax.experimental import pallas as pl


def kernel(input, emb_weight):
    raise NotImplementedError("write your pallas kernel here")


# speedup vs baseline: 1.8383x; 1.8383x over previous
import functools

import jax
import jax.numpy as jnp
from jax import lax
from jax.experimental import pallas as pl
from jax.experimental.pallas import tpu as pltpu
from jax.experimental.pallas import tpu_sc as plsc

LANES = 16


def _bow_encode(idx3, table, *, B, L, D, NC, NS):
    NW = NC * NS
    RPW = B // NW
    C = idx3.shape[2]
    NCHUNK = idx3.shape[1]
    DB = D // LANES

    mesh = plsc.VectorSubcoreMesh(core_axis_name="c", subcore_axis_name="s")

    @functools.partial(
        pl.kernel,
        mesh=mesh,
        out_type=jax.ShapeDtypeStruct((B, D), jnp.float32),
        compiler_params=pltpu.CompilerParams(use_tc_tiling_on_sc=False),
        scratch_types=[
            pltpu.VMEM((NCHUNK, C), jnp.int32),
            pltpu.VMEM((NCHUNK, C, D), jnp.float32),
            pltpu.VMEM((D,), jnp.float32),
            pltpu.SemaphoreType.DMA,
        ],
    )
    def run(idx_hbm, table_hbm, out_hbm, idx_v, rows_v, out_v, sem):
        wid = lax.axis_index("s") * NC + lax.axis_index("c")
        base = wid * RPW

        def row_body(r, carry):
            row = base + r
            pltpu.sync_copy(idx_hbm.at[row], idx_v)
            cps = [
                pltpu.async_copy(table_hbm.at[idx_v.at[c]], rows_v.at[c], sem)
                for c in range(NCHUNK)
            ]
            for cp in cps:
                cp.wait()

            accs = tuple(rows_v[0, 0, pl.ds(LANES * d, LANES)] for d in range(DB))

            def red(j, accs):
                out = []
                for d in range(DB):
                    a = accs[d]
                    for c in range(NCHUNK):
                        a = jnp.maximum(a, rows_v[c, j, pl.ds(LANES * d, LANES)])
                    out.append(a)
                return tuple(out)

            accs = lax.fori_loop(0, C, red, accs)
            for d in range(DB):
                out_v[pl.ds(LANES * d, LANES)] = accs[d]
            pltpu.sync_copy(out_v, out_hbm.at[row])
            return carry

        lax.fori_loop(0, RPW, row_body, 0)

    return run(idx3, table)


def kernel(input, emb_weight):
    B, L = input.shape
    V, D = emb_weight.shape
    NC, NS = 2, 16
    assert B % (NC * NS) == 0 and D % LANES == 0
    C = L // 2 if L % 2 == 0 and L // 2 <= 128 else L
    assert L % C == 0 and C <= 128
    idx3 = input.reshape(B, L // C, C)
    return _bow_encode(idx3, emb_weight, B=B, L=L, D=D, NC=NC, NS=NS)


# speedup vs baseline: 3.3500x; 1.8224x over previous
import functools

import jax
import jax.numpy as jnp
from jax import lax
from jax.experimental import pallas as pl
from jax.experimental.pallas import tpu as pltpu
from jax.experimental.pallas import tpu_sc as plsc

LANES = 16
NBUF = 4
BLK = 64
REDU = 4


def _bow_encode(idx3, table, *, B, D, NC, NS):
    NW = NC * NS
    RPW = B // NW
    NCHUNK, C = idx3.shape[1], idx3.shape[2]
    L = NCHUNK * C
    DB = D // LANES
    NBLK = RPW // BLK
    TGRP = BLK // NBUF

    mesh = plsc.VectorSubcoreMesh(core_axis_name="c", subcore_axis_name="s")

    @functools.partial(
        pl.kernel,
        mesh=mesh,
        out_type=jax.ShapeDtypeStruct((B, D), jnp.float32),
        compiler_params=pltpu.CompilerParams(use_tc_tiling_on_sc=False),
        scratch_types=[
            pltpu.VMEM((BLK, NCHUNK, C), jnp.int32),
            pltpu.VMEM((NBUF, L, D), jnp.float32),
            pltpu.VMEM((BLK, D), jnp.float32),
        ]
        + [pltpu.SemaphoreType.DMA] * NBUF,
    )
    def run(idx_hbm, table_hbm, out_hbm, idx_v, rows_v, out_v, *sems):
        wid = lax.axis_index("s") * NC + lax.axis_index("c")
        base = wid * RPW

        def gather_row(r_local, b, make_only):
            mk = pltpu.make_async_copy if make_only else pltpu.async_copy
            return [
                mk(table_hbm.at[idx_v.at[r_local, c]],
                   rows_v.at[b, pl.ds(c * C, C)], sems[b])
                for c in range(NCHUNK)
            ]

        def reduce_row(r_local, b):
            accs = tuple(rows_v[b, 0, pl.ds(LANES * d, LANES)]
                         for d in range(DB))

            def jbody(j, accs):
                out = []
                for d in range(DB):
                    a = accs[d]
                    for u in range(REDU):
                        a = jnp.maximum(
                            a, rows_v[b, j * REDU + u, pl.ds(LANES * d, LANES)])
                    out.append(a)
                return tuple(out)

            accs = lax.fori_loop(0, L // REDU, jbody, accs)
            for d in range(DB):
                out_v[r_local, pl.ds(LANES * d, LANES)] = accs[d]

        def blk_body(blk, carry):
            blk_base = base + blk * BLK
            pltpu.sync_copy(idx_hbm.at[pl.ds(blk_base, BLK)], idx_v)
            for b in range(NBUF):
                gather_row(b, b, False)

            def grp_body(t, carry):
                for b in range(NBUF):
                    r = t * NBUF + b
                    for cp in gather_row(r, b, True):
                        cp.wait()
                    reduce_row(r, b)

                    @pl.when(t < TGRP - 1)
                    def _():
                        gather_row(r + NBUF, b, False)
                return carry

            lax.fori_loop(0, TGRP, grp_body, 0)
            pltpu.sync_copy(out_v, out_hbm.at[pl.ds(blk_base, BLK)])
            return carry

        lax.fori_loop(0, NBLK, blk_body, 0)

    return run(idx3, table)


def kernel(input, emb_weight):
    B, L = input.shape
    V, D = emb_weight.shape
    NC, NS = 2, 16
    assert B % (NC * NS * BLK) == 0 and D % LANES == 0
    C = L // 2 if L % 2 == 0 and L // 2 <= 128 else L
    assert L % C == 0 and C <= 128 and L % REDU == 0
    idx3 = input.reshape(B, L // C, C)
    return _bow_encode(idx3, emb_weight, B=B, D=D, NC=NC, NS=NS)


# speedup vs baseline: 3.4204x; 1.0210x over previous
import functools

import jax
import jax.numpy as jnp
from jax import lax
from jax.experimental import pallas as pl
from jax.experimental.pallas import tpu as pltpu
from jax.experimental.pallas import tpu_sc as plsc

LANES = 16
NBUF = 4
BLK = 64
REDU = 4


def _bow_encode(idx, table, *, B, D, NC, NS):
    NW = NC * NS
    RPW = B // NW
    L = idx.shape[1]
    bounds = list(range(0, L, 128)) + [L]
    chunks = [(o, n - o) for o, n in zip(bounds[:-1], bounds[1:])]
    assert all(o % 8 == 0 and s % 8 == 0 and s <= 128 for o, s in chunks)
    DB = D // LANES
    NBLK = RPW // BLK
    TGRP = BLK // NBUF

    mesh = plsc.VectorSubcoreMesh(core_axis_name="c", subcore_axis_name="s")

    @functools.partial(
        pl.kernel,
        mesh=mesh,
        out_type=jax.ShapeDtypeStruct((B, D), jnp.float32),
        compiler_params=pltpu.CompilerParams(use_tc_tiling_on_sc=False),
        scratch_types=[
            pltpu.VMEM((BLK, L), jnp.int32),
            pltpu.VMEM((NBUF, L, D), jnp.float32),
            pltpu.VMEM((BLK, D), jnp.float32),
        ]
        + [pltpu.SemaphoreType.DMA] * NBUF,
    )
    def run(idx_hbm, table_hbm, out_hbm, idx_v, rows_v, out_v, *sems):
        wid = lax.axis_index("s") * NC + lax.axis_index("c")
        base = wid * RPW

        def gather_row(r_local, b, make_only):
            mk = pltpu.make_async_copy if make_only else pltpu.async_copy
            return [
                mk(table_hbm.at[idx_v.at[r_local, pl.ds(o, s)]],
                   rows_v.at[b, pl.ds(o, s)], sems[b])
                for o, s in chunks
            ]

        def reduce_row(r_local, b):
            accs = tuple(rows_v[b, 0, pl.ds(LANES * d, LANES)]
                         for d in range(DB))

            def jbody(j, accs):
                out = []
                for d in range(DB):
                    a = accs[d]
                    for u in range(REDU):
                        a = jnp.maximum(
                            a, rows_v[b, j * REDU + u, pl.ds(LANES * d, LANES)])
                    out.append(a)
                return tuple(out)

            accs = lax.fori_loop(0, L // REDU, jbody, accs)
            for d in range(DB):
                out_v[r_local, pl.ds(LANES * d, LANES)] = accs[d]

        def blk_body(blk, carry):
            blk_base = base + blk * BLK
            pltpu.sync_copy(idx_hbm.at[pl.ds(blk_base, BLK)], idx_v)
            for b in range(NBUF):
                gather_row(b, b, False)

            def grp_body(t, carry):
                for b in range(NBUF):
                    r = t * NBUF + b
                    for cp in gather_row(r, b, True):
                        cp.wait()
                    reduce_row(r, b)

                    @pl.when(t < TGRP - 1)
                    def _():
                        gather_row(r + NBUF, b, False)
                return carry

            lax.fori_loop(0, TGRP, grp_body, 0)
            pltpu.sync_copy(out_v, out_hbm.at[pl.ds(blk_base, BLK)])
            return carry

        lax.fori_loop(0, NBLK, blk_body, 0)

    return run(idx, table)


def kernel(input, emb_weight):
    B, L = input.shape
    V, D = emb_weight.shape
    NC, NS = 2, 16
    assert B % (NC * NS * BLK) == 0 and D % LANES == 0 and L % REDU == 0
    return _bow_encode(input, emb_weight, B=B, D=D, NC=NC, NS=NS)
